# Initial kernel scaffold; baseline (speedup 1.0000x reference)
import jax
import jax.numpy as jnp
from jax.experimental import pallas as pl


def kernel(x, edge_index, W, b):
    raise NotImplementedError("write your pallas kernel here")


# speedup vs baseline: 5.6091x; 5.6091x over previous
import functools

import jax
import jax.numpy as jnp
from jax import lax
from jax.experimental import pallas as pl
from jax.experimental.pallas import tpu as pltpu
from jax.experimental.pallas import tpu_sc as plsc

NC = 2
NS = 16
LN = 16
NW = NC * NS


def _mesh():
    return plsc.VectorSubcoreMesh(
        core_axis_name="c", subcore_axis_name="s",
        num_cores=NC, num_subcores=NS)


def _deg_body(n_pad, epw, chunk, src_ref, dst_ref, dego_ref, degi_ref,
              idx_s, idx_d, hist_o, hist_i, red_buf, acc_buf,
              shared_o, shared_i):
    cid = lax.axis_index("c")
    sid = lax.axis_index("s")
    wid = sid * NC + cid
    ones = jnp.ones((LN,), jnp.float32)
    zeros = jnp.zeros((LN,), jnp.float32)

    def zero_hist(i, _):
        hist_o[pl.ds(i * LN, LN)] = zeros
        hist_i[pl.ds(i * LN, LN)] = zeros
        return 0
    lax.fori_loop(0, n_pad // LN, zero_hist, 0)

    base = wid * epw

    def do_chunk(c, _):
        off = base + c * chunk
        pltpu.sync_copy(src_ref.at[pl.ds(off, chunk)], idx_s)
        pltpu.sync_copy(dst_ref.at[pl.ds(off, chunk)], idx_d)
        for j in range(chunk // LN):
            si = idx_s[pl.ds(j * LN, LN)]
            plsc.addupdate_scatter(hist_o, [si], ones)
            di = idx_d[pl.ds(j * LN, LN)]
            plsc.addupdate_scatter(hist_i, [di], ones)
        return 0
    lax.fori_loop(0, epw // chunk, do_chunk, 0)

    pltpu.sync_copy(hist_o, shared_o.at[sid])
    pltpu.sync_copy(hist_i, shared_i.at[sid])
    plsc.subcore_barrier()

    cpt = n_pad // NS
    col0 = sid * cpt
    for which in range(2):
        sh = shared_o if which == 0 else shared_i
        dst = dego_ref if which == 0 else degi_ref
        pltpu.sync_copy(sh.at[:, pl.ds(col0, cpt)], red_buf)

        def red(v, _):
            acc = red_buf[0, pl.ds(v * LN, LN)]
            for r in range(1, NS):
                acc = acc + red_buf[r, pl.ds(v * LN, LN)]
            acc_buf[pl.ds(v * LN, LN)] = acc
            return 0
        lax.fori_loop(0, cpt // LN, red, 0)
        pltpu.sync_copy(acc_buf, dst.at[cid, pl.ds(col0, cpt)])


def _make_deg_call(e, n_pad):
    epw = e // NW
    chunk = 400
    assert epw % chunk == 0 and chunk % LN == 0
    cpt = n_pad // NS
    body = functools.partial(_deg_body, n_pad, epw, chunk)
    return pl.kernel(
        body,
        out_type=(jax.ShapeDtypeStruct((NC, n_pad), jnp.float32),
                  jax.ShapeDtypeStruct((NC, n_pad), jnp.float32)),
        mesh=_mesh(),
        scratch_types=[
            pltpu.VMEM((chunk,), jnp.int32),
            pltpu.VMEM((chunk,), jnp.int32),
            pltpu.VMEM((n_pad,), jnp.float32),
            pltpu.VMEM((n_pad,), jnp.float32),
            pltpu.VMEM((NS, cpt), jnp.float32),
            pltpu.VMEM((cpt,), jnp.float32),
            pltpu.VMEM_SHARED((NS, n_pad), jnp.float32),
            pltpu.VMEM_SHARED((NS, n_pad), jnp.float32),
        ],
        compiler_params=pltpu.CompilerParams(needs_layout_passes=False),
    )


def _agg_body(n_pad, d, epw, chunk, h_ref, src_ref, dst_ref, out_ref,
              idx_s, idx_d, rows, zbuf, acc_sh, sem):
    cid = lax.axis_index("c")
    sid = lax.axis_index("s")
    wid = sid * NC + cid

    zrows = zbuf.shape[0]

    def zfill(r, _):
        for k in range(d // LN):
            zbuf[r, pl.ds(k * LN, LN)] = jnp.zeros((LN,), jnp.float32)
        return 0
    lax.fori_loop(0, zrows, zfill, 0)
    rpt = n_pad // NS
    for k in range(rpt // zrows):
        pltpu.sync_copy(zbuf, acc_sh.at[pl.ds(sid * rpt + k * zrows, zrows)])
    plsc.subcore_barrier()

    base = wid * epw

    def do_chunk(c, _):
        off = base + c * chunk
        pltpu.sync_copy(src_ref.at[pl.ds(off, chunk)], idx_s)
        pltpu.sync_copy(dst_ref.at[pl.ds(off, chunk)], idx_d)
        pltpu.async_copy(h_ref.at[idx_s], rows, sem).wait()
        pltpu.sync_copy(rows, acc_sh.at[idx_d], add=True)
        return 0
    lax.fori_loop(0, epw // chunk, do_chunk, 0)
    plsc.subcore_barrier()

    for k in range(rpt // zrows):
        r0 = sid * rpt + k * zrows
        pltpu.sync_copy(acc_sh.at[pl.ds(r0, zrows)],
                        out_ref.at[cid, pl.ds(r0, zrows)])


def _make_agg_call(n_pad, d, e):
    epw = e // NW
    chunk = 80
    assert epw % chunk == 0
    rpt = n_pad // NS
    zrows = 128
    assert rpt % zrows == 0
    body = functools.partial(_agg_body, n_pad, d, epw, chunk)
    return pl.kernel(
        body,
        out_type=jax.ShapeDtypeStruct((NC, n_pad, d), jnp.float32),
        mesh=_mesh(),
        scratch_types=[
            pltpu.VMEM((chunk,), jnp.int32),
            pltpu.VMEM((chunk,), jnp.int32),
            pltpu.VMEM((chunk, d), jnp.float32),
            pltpu.VMEM((zrows, d), jnp.float32),
            pltpu.VMEM_SHARED((n_pad, d), jnp.float32),
            pltpu.SemaphoreType.DMA,
        ],
        compiler_params=pltpu.CompilerParams(needs_layout_passes=False),
    )


def _mm_body(x_ref, deg_ref, w_ref, out_ref):
    dpair = deg_ref[...]
    dsum = dpair[:, 0] + dpair[:, 1]
    ns = lax.rsqrt(jnp.maximum(dsum, 1.0))
    h = x_ref[...] * ns[:, None]
    out_ref[...] = jnp.dot(h, w_ref[...], preferred_element_type=jnp.float32)


def _make_mm_call(n, d, rows):
    assert n % rows == 0
    grid = n // rows
    return pl.pallas_call(
        _mm_body,
        grid=(grid,),
        in_specs=[
            pl.BlockSpec((rows, d), lambda i: (i, 0)),
            pl.BlockSpec((rows, NC), lambda i: (i, 0)),
            pl.BlockSpec((d, d), lambda i: (0, 0)),
        ],
        out_specs=pl.BlockSpec((rows, d), lambda i: (i, 0)),
        out_shape=jax.ShapeDtypeStruct((n, d), jnp.float32),
    )


def _fin_body(p_ref, deg_ref, b_ref, out_ref):
    agg = p_ref[0] + p_ref[1]
    dpair = deg_ref[...]
    dsum = dpair[:, 0] + dpair[:, 1]
    nd = lax.rsqrt(jnp.maximum(dsum, 1.0))
    out = agg * nd[:, None] + b_ref[...]
    out_ref[...] = jnp.maximum(out, 0.0)


def _make_fin_call(n, d, rows):
    assert n % rows == 0
    grid = n // rows
    return pl.pallas_call(
        _fin_body,
        grid=(grid,),
        in_specs=[
            pl.BlockSpec((NC, rows, d), lambda i: (0, i, 0)),
            pl.BlockSpec((rows, NC), lambda i: (i, 0)),
            pl.BlockSpec((1, d), lambda i: (0, 0)),
        ],
        out_specs=pl.BlockSpec((rows, d), lambda i: (i, 0)),
        out_shape=jax.ShapeDtypeStruct((n, d), jnp.float32),
    )


def kernel(x, edge_index, W, b):
    n, d = x.shape
    e = edge_index.shape[1]
    n_pad = ((n + (LN * NS) - 1) // (LN * NS)) * (LN * NS)

    src = edge_index[0]
    dst = edge_index[1]
    dego_p, degi_p = _make_deg_call(e, n_pad)(src, dst)
    dego = jnp.transpose(dego_p[:, :n])
    degi = jnp.transpose(degi_p[:, :n])

    hw = _make_mm_call(n, d, 1000)(x, dego, W)
    parts = _make_agg_call(n_pad, d, e)(hw, src, dst)[:, :n, :]
    out = _make_fin_call(n, d, 1000)(parts, degi, jnp.reshape(b, (1, d)))
    return out
